# baseline (speedup 1.0000x reference)
import jax
import jax.numpy as jnp
from jax.experimental import pallas as pl
from jax.experimental.pallas import tpu as pltpu

_ROWS = 32768
_W = 512
_CR = 1024
_NSTEP = _ROWS // _CR
_NBUF = 2


def _body(pred_hbm, meas_hbm, mask_hbm, out_hbm,
          pred_b, meas_b, mask_b, out_b,
          in_sems, out_sems):
    i = pl.program_id(0)

    def start_in(step, slot):
        sl = pl.ds(step * _CR, _CR)
        pltpu.make_async_copy(pred_hbm.at[sl], pred_b.at[slot], in_sems.at[slot, 0]).start()
        pltpu.make_async_copy(meas_hbm.at[sl], meas_b.at[slot], in_sems.at[slot, 1]).start()
        pltpu.make_async_copy(mask_hbm.at[sl], mask_b.at[slot], in_sems.at[slot, 2]).start()

    @pl.when(i == 0)
    def _prologue():
        start_in(0, 0)

    slot = jax.lax.rem(i, jnp.int32(_NBUF))
    nxt = jax.lax.rem(i + 1, jnp.int32(_NBUF))

    @pl.when(i + 1 < _NSTEP)
    def _prefetch():
        start_in(i + 1, nxt)

    sl = pl.ds(i * _CR, _CR)
    pltpu.make_async_copy(pred_hbm.at[sl], pred_b.at[slot], in_sems.at[slot, 0]).wait()
    pltpu.make_async_copy(meas_hbm.at[sl], meas_b.at[slot], in_sems.at[slot, 1]).wait()
    pltpu.make_async_copy(mask_hbm.at[sl], mask_b.at[slot], in_sems.at[slot, 2]).wait()

    @pl.when(i >= _NBUF)
    def _drain_prev():
        prev = pl.ds((i - _NBUF) * _CR, _CR)
        pltpu.make_async_copy(out_b.at[slot], out_hbm.at[prev], out_sems.at[slot]).wait()

    out_b[slot] = jnp.where(mask_b[slot] != 0, meas_b[slot], pred_b[slot])

    pltpu.make_async_copy(out_b.at[slot], out_hbm.at[sl], out_sems.at[slot]).start()

    @pl.when(i == _NSTEP - 1)
    def _epilogue():
        for k in range(_NBUF):
            step = _NSTEP - _NBUF + k
            s2 = jax.lax.rem(jnp.int32(step), jnp.int32(_NBUF))
            pltpu.make_async_copy(
                out_b.at[s2], out_hbm.at[pl.ds(step * _CR, _CR)], out_sems.at[s2]
            ).wait()


def kernel(k_pred, k_meas, mask):
    B, H, W = k_pred.shape
    mask8 = mask.view(jnp.int8).reshape(_ROWS, _W)
    out = pl.pallas_call(
        _body,
        grid=(_NSTEP,),
        in_specs=[pl.BlockSpec(memory_space=pltpu.MemorySpace.HBM)] * 3,
        out_specs=pl.BlockSpec(memory_space=pltpu.MemorySpace.HBM),
        out_shape=jax.ShapeDtypeStruct((_ROWS, _W), jnp.float32),
        scratch_shapes=[
            pltpu.VMEM((_NBUF, _CR, _W), jnp.float32),
            pltpu.VMEM((_NBUF, _CR, _W), jnp.float32),
            pltpu.VMEM((_NBUF, _CR, _W), jnp.int8),
            pltpu.VMEM((_NBUF, _CR, _W), jnp.float32),
            pltpu.SemaphoreType.DMA((_NBUF, 3)),
            pltpu.SemaphoreType.DMA((_NBUF,)),
        ],
    )(k_pred.reshape(_ROWS, _W), k_meas.reshape(_ROWS, _W), mask8)
    return out.reshape(B, H, W)


# speedup vs baseline: 1.0170x; 1.0170x over previous
import jax
import jax.numpy as jnp
from jax.experimental import pallas as pl
from jax.experimental.pallas import tpu as pltpu


def _dc_block(pred_ref, meas_ref, mask_ref, out_ref):
    out_ref[...] = jnp.where(mask_ref[...] != 0, meas_ref[...], pred_ref[...])


def kernel(k_pred, k_meas, mask):
    B, H, W = k_pred.shape
    mask8 = mask.view(jnp.int8)
    blk = 4
    specs = [pl.BlockSpec((blk, H, W), lambda i: (i, 0, 0)) for _ in range(3)]
    return pl.pallas_call(
        _dc_block,
        grid=(B // blk,),
        in_specs=specs,
        out_specs=pl.BlockSpec((blk, H, W), lambda i: (i, 0, 0)),
        out_shape=jax.ShapeDtypeStruct((B, H, W), jnp.float32),
        compiler_params=pltpu.CompilerParams(
            dimension_semantics=("parallel",),
        ),
    )(k_pred, k_meas, mask8)
